# baseline (speedup 1.0000x reference)
import functools

import jax
import jax.numpy as jnp
from jax import lax
from jax.experimental import pallas as pl
from jax.experimental.pallas import tpu as pltpu
from jax.experimental.pallas import tpu_sc as plsc

B, C, D = 4, 4096, 1024
NC, NS = 2, 16
NW = NC * NS
C_PER_W = C // NW
RC = 8
NCHUNK = C_PER_W // RC
CH = RC * D
NVEC = CH // 16


def _sc_body(x_hbm, emb_hbm, out_hbm, ebuf, xbuf, sem_in0, sem_in1, sem_out):
    cid = lax.axis_index("c")
    sid = lax.axis_index("s")
    w = sid * NC + cid
    c0 = w * C_PER_W
    sems = (sem_in0, sem_in1)

    def fire_loads(k, p):
        base = pl.multiple_of((c0 + k * RC) * D, CH)
        cps = [pltpu.async_copy(emb_hbm.at[pl.ds(base, CH)], ebuf.at[p], sems[p])]
        for b in range(B):
            cps.append(pltpu.async_copy(
                x_hbm.at[pl.ds(b * (C * D) + base, CH)], xbuf.at[p, b], sems[p]))
        return cps

    def fire_stores(k, p):
        base = pl.multiple_of((c0 + k * RC) * D, CH)
        return [pltpu.async_copy(
            xbuf.at[p, b], out_hbm.at[pl.ds(b * (C * D) + base, CH)], sem_out)
            for b in range(B)]

    loads = fire_loads(0, 0)
    stores = []
    for k in range(NCHUNK):
        p = k % 2
        if k + 1 < NCHUNK:
            for cp in stores:
                cp.wait()
            next_loads = fire_loads(k + 1, (k + 1) % 2)
        else:
            next_loads = []
        for cp in loads:
            cp.wait()

        def _add(j, c2):
            for u in range(8):
                off = (j * 8 + u) * 16
                e = ebuf[p, pl.ds(off, 16)]
                for b in range(B):
                    xv = xbuf[p, b, pl.ds(off, 16)]
                    xbuf[p, b, pl.ds(off, 16)] = xv + e
            return c2

        lax.fori_loop(0, NVEC // 8, _add, 0)

        stores = fire_stores(k, p)
        loads = next_loads
    for cp in stores:
        cp.wait()


@functools.partial(
    pl.kernel,
    mesh=plsc.VectorSubcoreMesh(core_axis_name="c", subcore_axis_name="s"),
    out_type=jax.ShapeDtypeStruct((B * C * D,), jnp.float32),
    scratch_types=[
        pltpu.VMEM((2, CH), jnp.float32),
        pltpu.VMEM((2, B, CH), jnp.float32),
        pltpu.SemaphoreType.DMA,
        pltpu.SemaphoreType.DMA,
        pltpu.SemaphoreType.DMA,
    ],
)
def _sc_kernel(x_hbm, emb_hbm, out_hbm, ebuf, xbuf, sem_in0, sem_in1, sem_out):
    _sc_body(x_hbm, emb_hbm, out_hbm, ebuf, xbuf, sem_in0, sem_in1, sem_out)


def kernel(x, embedding):
    b, c, d = x.shape
    out = _sc_kernel(x.reshape(-1), embedding.reshape(-1))
    return out.reshape(b, c, d)


# speedup vs baseline: 1.0300x; 1.0300x over previous
import functools

import jax
import jax.numpy as jnp
from jax import lax
from jax.experimental import pallas as pl
from jax.experimental.pallas import tpu as pltpu
from jax.experimental.pallas import tpu_sc as plsc

B, C, D = 4, 4096, 1024
NC, NS = 2, 16
NW = NC * NS
C_PER_W = C // NW
RC = 4
NCHUNK = C_PER_W // RC
NPAIR = NCHUNK // 2
CH = RC * D
NVEC = CH // 16


def _sc_body(x_hbm, emb_hbm, out_hbm, ebuf, xbuf, obuf,
             sem_in0, sem_in1, sem_out0, sem_out1):
    cid = lax.axis_index("c")
    sid = lax.axis_index("s")
    w = sid * NC + cid
    c0 = w * C_PER_W
    sems_in = (sem_in0, sem_in1)
    sems_out = (sem_out0, sem_out1)

    def load_copies(k, p):
        base = pl.multiple_of((c0 + k * RC) * D, CH)
        cps = [pltpu.make_async_copy(emb_hbm.at[pl.ds(base, CH)], ebuf.at[p],
                                     sems_in[p])]
        for b in range(B):
            cps.append(pltpu.make_async_copy(
                x_hbm.at[pl.ds(b * (C * D) + base, CH)], xbuf.at[p, b],
                sems_in[p]))
        return cps

    def store_copies(k, p):
        base = pl.multiple_of((c0 + k * RC) * D, CH)
        return [pltpu.make_async_copy(
            obuf.at[p, b], out_hbm.at[pl.ds(b * (C * D) + base, CH)],
            sems_out[p])
            for b in range(B)]

    for cp in load_copies(0, 0) + load_copies(1, 1):
        cp.start()

    def pair(kk, carry):
        for p in range(2):
            k = 2 * kk + p
            for cp in load_copies(k, p):
                cp.wait()

            @pl.when(kk >= 1)
            def _drain():
                for cp in store_copies(k - 2, p):
                    cp.wait()

            def _add(j, c2):
                for u in range(8):
                    off = (j * 8 + u) * 16
                    e = ebuf[p, pl.ds(off, 16)]
                    for b in range(B):
                        obuf[p, b, pl.ds(off, 16)] = (
                            xbuf[p, b, pl.ds(off, 16)] + e)
                return c2

            lax.fori_loop(0, NVEC // 8, _add, 0)

            for cp in store_copies(k, p):
                cp.start()

            @pl.when(kk + 1 < NPAIR)
            def _prefetch():
                for cp in load_copies(k + 2, p):
                    cp.start()
        return carry

    lax.fori_loop(0, NPAIR, pair, 0)

    for p in range(2):
        for cp in store_copies(NCHUNK - 2 + p, p):
            cp.wait()


@functools.partial(
    pl.kernel,
    mesh=plsc.VectorSubcoreMesh(core_axis_name="c", subcore_axis_name="s"),
    out_type=jax.ShapeDtypeStruct((B * C * D,), jnp.float32),
    scratch_types=[
        pltpu.VMEM((2, CH), jnp.float32),
        pltpu.VMEM((2, B, CH), jnp.float32),
        pltpu.VMEM((2, B, CH), jnp.float32),
        pltpu.SemaphoreType.DMA,
        pltpu.SemaphoreType.DMA,
        pltpu.SemaphoreType.DMA,
        pltpu.SemaphoreType.DMA,
    ],
)
def _sc_kernel(x_hbm, emb_hbm, out_hbm, ebuf, xbuf, obuf,
               sem_in0, sem_in1, sem_out0, sem_out1):
    _sc_body(x_hbm, emb_hbm, out_hbm, ebuf, xbuf, obuf,
             sem_in0, sem_in1, sem_out0, sem_out1)


def kernel(x, embedding):
    b, c, d = x.shape
    out = _sc_kernel(x.reshape(-1), embedding.reshape(-1))
    return out.reshape(b, c, d)


# speedup vs baseline: 3.6520x; 3.5456x over previous
import functools

import jax
import jax.numpy as jnp
from jax import lax
from jax.experimental import pallas as pl
from jax.experimental.pallas import tpu as pltpu
from jax.experimental.pallas import tpu_sc as plsc

B, C, D = 4, 4096, 1024
NC, NS = 2, 16
NW = NC * NS
C_PER_W = C // NW
RC = 4
NCHUNK = C_PER_W // RC
NPAIR = NCHUNK // 2
NVJ = D // (16 * 8)


def _sc_body(x_hbm, emb_hbm, out_hbm, ebuf, xbuf, obuf, sem_in0, sem_in1,
             sem_out0, sem_out1):
    cid = lax.axis_index("c")
    sid = lax.axis_index("s")
    w = sid * NC + cid
    c0 = w * C_PER_W
    sems_in = (sem_in0, sem_in1)
    sems_out = (sem_out0, sem_out1)

    def load_copies(k, p):
        lo = c0 + k * RC
        cps = [pltpu.make_async_copy(emb_hbm.at[pl.ds(lo, RC)], ebuf.at[p],
                                     sems_in[p])]
        for b in range(B):
            cps.append(pltpu.make_async_copy(
                x_hbm.at[b, pl.ds(lo, RC)], xbuf.at[p, b], sems_in[p]))
        return cps

    def store_copies(k, p):
        lo = c0 + k * RC
        return [pltpu.make_async_copy(
            obuf.at[p, b], out_hbm.at[b, pl.ds(lo, RC)], sems_out[p])
            for b in range(B)]

    for cp in load_copies(0, 0) + load_copies(1, 1):
        cp.start()

    def pair(kk, carry):
        for p in range(2):
            k = 2 * kk + p
            for cp in load_copies(k, p):
                cp.wait()

            @pl.when(kk >= 1)
            def _drain():
                for cp in store_copies(k - 2, p):
                    cp.wait()

            for r in range(RC):
                def _add(j, c2, r=r):
                    for u in range(8):
                        off = (j * 8 + u) * 16
                        e = ebuf[p, r, pl.ds(off, 16)]
                        for b in range(B):
                            obuf[p, b, r, pl.ds(off, 16)] = (
                                xbuf[p, b, r, pl.ds(off, 16)] + e)
                    return c2

                lax.fori_loop(0, NVJ, _add, 0)

            for cp in store_copies(k, p):
                cp.start()

            @pl.when(kk + 1 < NPAIR)
            def _prefetch():
                for cp in load_copies(k + 2, p):
                    cp.start()
        return carry

    lax.fori_loop(0, NPAIR, pair, 0)

    for p in range(2):
        for cp in store_copies(NCHUNK - 2 + p, p):
            cp.wait()


@functools.partial(
    pl.kernel,
    mesh=plsc.VectorSubcoreMesh(core_axis_name="c", subcore_axis_name="s"),
    out_type=jax.ShapeDtypeStruct((B, C, D), jnp.float32),
    scratch_types=[
        pltpu.VMEM((2, RC, D), jnp.float32),
        pltpu.VMEM((2, B, RC, D), jnp.float32),
        pltpu.VMEM((2, B, RC, D), jnp.float32),
        pltpu.SemaphoreType.DMA,
        pltpu.SemaphoreType.DMA,
        pltpu.SemaphoreType.DMA,
        pltpu.SemaphoreType.DMA,
    ],
)
def _sc_kernel(x_hbm, emb_hbm, out_hbm, ebuf, xbuf, obuf, sem_in0, sem_in1,
               sem_out0, sem_out1):
    _sc_body(x_hbm, emb_hbm, out_hbm, ebuf, xbuf, obuf, sem_in0, sem_in1,
             sem_out0, sem_out1)


def kernel(x, embedding):
    return _sc_kernel(x, embedding)


# speedup vs baseline: 3.6664x; 1.0039x over previous
import functools

import jax
import jax.numpy as jnp
from jax import lax
from jax.experimental import pallas as pl
from jax.experimental.pallas import tpu as pltpu
from jax.experimental.pallas import tpu_sc as plsc

B, C, D = 4, 4096, 1024
NC, NS = 2, 16
NW = NC * NS
C_PER_W = C // NW
RC = 4
NCHUNK = C_PER_W // RC
NPAIR = NCHUNK // 2
NVJ = D // (16 * 8)


def _sc_body(x_hbm, emb_hbm, out_hbm, ebuf, xbuf, obuf, sem_in0, sem_in1,
             sem_out0, sem_out1):
    cid = lax.axis_index("c")
    sid = lax.axis_index("s")
    w = sid * NC + cid
    c0 = w * C_PER_W
    sems_in = (sem_in0, sem_in1)
    sems_out = (sem_out0, sem_out1)

    def load_copies(k, p):
        lo = c0 + k * RC
        return [
            pltpu.make_async_copy(emb_hbm.at[pl.ds(lo, RC)], ebuf.at[p],
                                  sems_in[p]),
            pltpu.make_async_copy(x_hbm.at[:, pl.ds(lo, RC)], xbuf.at[p],
                                  sems_in[p]),
        ]

    def store_copies(k, p):
        lo = c0 + k * RC
        return [pltpu.make_async_copy(
            obuf.at[p], out_hbm.at[:, pl.ds(lo, RC)], sems_out[p])]

    for cp in load_copies(0, 0) + load_copies(1, 1):
        cp.start()

    def pair(kk, carry):
        for p in range(2):
            k = 2 * kk + p
            for cp in load_copies(k, p):
                cp.wait()

            @pl.when(kk >= 1)
            def _drain():
                for cp in store_copies(k - 2, p):
                    cp.wait()

            for r in range(RC):
                def _add(j, c2, r=r):
                    for u in range(8):
                        off = (j * 8 + u) * 16
                        e = ebuf[p, r, pl.ds(off, 16)]
                        for b in range(B):
                            obuf[p, b, r, pl.ds(off, 16)] = (
                                xbuf[p, b, r, pl.ds(off, 16)] + e)
                    return c2

                lax.fori_loop(0, NVJ, _add, 0)

            for cp in store_copies(k, p):
                cp.start()

            @pl.when(kk + 1 < NPAIR)
            def _prefetch():
                for cp in load_copies(k + 2, p):
                    cp.start()
        return carry

    lax.fori_loop(0, NPAIR, pair, 0)

    for p in range(2):
        for cp in store_copies(NCHUNK - 2 + p, p):
            cp.wait()


@functools.partial(
    pl.kernel,
    mesh=plsc.VectorSubcoreMesh(core_axis_name="c", subcore_axis_name="s"),
    out_type=jax.ShapeDtypeStruct((B, C, D), jnp.float32),
    scratch_types=[
        pltpu.VMEM((2, RC, D), jnp.float32),
        pltpu.VMEM((2, B, RC, D), jnp.float32),
        pltpu.VMEM((2, B, RC, D), jnp.float32),
        pltpu.SemaphoreType.DMA,
        pltpu.SemaphoreType.DMA,
        pltpu.SemaphoreType.DMA,
        pltpu.SemaphoreType.DMA,
    ],
)
def _sc_kernel(x_hbm, emb_hbm, out_hbm, ebuf, xbuf, obuf, sem_in0, sem_in1,
               sem_out0, sem_out1):
    _sc_body(x_hbm, emb_hbm, out_hbm, ebuf, xbuf, obuf, sem_in0, sem_in1,
             sem_out0, sem_out1)


def kernel(x, embedding):
    return _sc_kernel(x, embedding)
